# Initial kernel scaffold; baseline (speedup 1.0000x reference)
import jax
import jax.numpy as jnp
from jax.experimental import pallas as pl


def kernel(x, edge_index, W1, b1, W2, b2, Wmu, bmu, Wlv, blv):
    raise NotImplementedError("write your pallas kernel here")


# speedup vs baseline: 13.1404x; 13.1404x over previous
import functools

import jax
import jax.numpy as jnp
from jax import lax
from jax.experimental import pallas as pl
from jax.experimental.pallas import tpu as pltpu
from jax.experimental.pallas import tpu_sc as plsc

N = 10000
E = 160000
D_IN = 256
D_H = 256
D_Z = 64

NC = 2
NS = 16
NP = 10240
RT = NP // NS
NPT = NP // NS

K = 80
NCH = (E // NS) // K
ED = E // (NC * NS)
K3 = 100
NCH3 = ED // K3

_MESH = dict(mesh=plsc.VectorSubcoreMesh(core_axis_name="c", subcore_axis_name="s"))


def _deg_body(dd_hbm, out_hbm, ebuf, hist, cbuf, obuf, hshared):
    c = lax.axis_index("c")
    s = lax.axis_index("s")
    w = c * NS + s

    zeros = jnp.zeros((16,), jnp.float32)

    def zbody(j, carry):
        hist[pl.ds(j * 16, 16)] = zeros
        return carry

    lax.fori_loop(0, NP // 16, zbody, 0)

    pltpu.sync_copy(dd_hbm.at[pl.ds(w * ED, ED)], ebuf)
    ones = jnp.ones((16,), jnp.float32)

    def hbody(i, carry):
        idx = ebuf[pl.ds(i * 16, 16)]
        plsc.addupdate_scatter(hist, [idx], ones)
        return carry

    full = ED // 16
    lax.fori_loop(0, full, hbody, 0)
    idx = ebuf[pl.ds(ED - 16, 16)]
    rem = lax.iota(jnp.int32, 16) >= (16 - (ED - full * 16))
    plsc.addupdate_scatter(hist, [idx], ones, mask=rem)

    pltpu.sync_copy(hist, hshared.at[pl.ds(s * NP, NP)])
    plsc.subcore_barrier()
    for p in range(NS):
        pltpu.sync_copy(hshared.at[pl.ds(p * NP + s * RT, RT)],
                        cbuf.at[pl.ds(p * RT, RT)])

    def rbody(j, carry):
        v = cbuf[pl.ds(j * 16, 16)]
        for p in range(1, NS):
            v = v + cbuf[pl.ds(p * RT + j * 16, 16)]
        obuf[pl.ds(j * 16, 16)] = v
        return carry

    lax.fori_loop(0, RT // 16, rbody, 0)
    pltpu.sync_copy(obuf, out_hbm.at[pl.ds(w * RT, RT)])


@functools.partial(
    pl.kernel,
    out_type=jax.ShapeDtypeStruct((NC * NS * RT,), jnp.float32),
    scratch_types=[
        pltpu.VMEM((ED,), jnp.int32),
        pltpu.VMEM((NP,), jnp.float32),
        pltpu.VMEM((NS * RT,), jnp.float32),
        pltpu.VMEM((RT,), jnp.float32),
        pltpu.VMEM_SHARED((NS * NP,), jnp.float32),
    ],
    compiler_params=pltpu.CompilerParams(needs_layout_passes=False),
    **_MESH,
)
def _sc_degree(dd_hbm, out_hbm, ebuf, hist, cbuf, obuf, hshared):
    _deg_body(dd_hbm, out_hbm, ebuf, hist, cbuf, obuf, hshared)


def _pass_body(nch, t_hbm, so_hbm, d_hbm, out_hbm, idxs, idxd, rows, acc):
    c = lax.axis_index("c")
    s = lax.axis_index("s")

    pltpu.sync_copy(t_hbm.at[pl.ds(c * NP + s * NPT, NPT)],
                    acc.at[pl.ds(s * NPT, NPT)])
    pltpu.sync_copy(so_hbm.at[c, s], idxs)
    pltpu.sync_copy(d_hbm.at[c, s], idxd)
    plsc.subcore_barrier()

    def body(g, carry):
        pltpu.sync_copy(t_hbm.at[idxs.at[g, 0]], rows)
        pltpu.sync_copy(rows, acc.at[idxd.at[g, 0]], add=True)
        return carry

    lax.fori_loop(0, nch, body, 0)
    plsc.subcore_barrier()
    pltpu.sync_copy(acc.at[pl.ds(s * NPT, NPT)],
                    out_hbm.at[c, pl.ds(s * NPT, NPT)])


def _make_sc_pass(dh, nch, k):
    @functools.partial(
        pl.kernel,
        out_type=jax.ShapeDtypeStruct((NC, NP, dh), jnp.float32),
        scratch_types=[
            pltpu.VMEM((nch, 1, k), jnp.int32),
            pltpu.VMEM((nch, 1, k), jnp.int32),
            pltpu.VMEM((k, dh), jnp.float32),
            pltpu.VMEM_SHARED((NP, dh), jnp.float32),
        ],
        **_MESH,
    )
    def _sc_pass(t_hbm, so_hbm, d_hbm, out_hbm, idxs, idxd, rows, acc):
        _pass_body(nch, t_hbm, so_hbm, d_hbm, out_hbm, idxs, idxd, rows, acc)

    return _sc_pass


_sc_pass_cols = _make_sc_pass(D_H // 2, NCH, K)
_sc_pass_edges = _make_sc_pass(2 * D_Z, NCH3, K3)


R = 2000
GRID = N // R

_SPEC_DEG = pl.BlockSpec((R, 2), lambda i: (i, 0))


def _dis(deg_ref):
    d = deg_ref[...]
    return lax.rsqrt(d[:, 0:1] + d[:, 1:2] + 1.0)


def _tc1_body(x_ref, deg_ref, w_ref, out_ref):
    dis = _dis(deg_ref)
    y = dis * jnp.dot(x_ref[...], w_ref[...],
                      preferred_element_type=jnp.float32,
                      precision=lax.Precision.HIGHEST)
    out_ref[0, :, :] = y[:, :D_H // 2]
    out_ref[1, :, :] = y[:, D_H // 2:]


def _tc1(x, degT, W1):
    return pl.pallas_call(
        _tc1_body,
        grid=(GRID,),
        in_specs=[
            pl.BlockSpec((R, D_IN), lambda i: (i, 0)),
            _SPEC_DEG,
            pl.BlockSpec((D_IN, D_H), lambda i: (0, 0)),
        ],
        out_specs=pl.BlockSpec((2, R, D_H // 2), lambda i: (0, i, 0)),
        out_shape=jax.ShapeDtypeStruct((2, NP, D_H // 2), jnp.float32),
    )(x, degT, W1)


def _tc2_body(s_ref, deg_ref, b_ref, w_ref, out_ref):
    dis = _dis(deg_ref)
    scat = jnp.concatenate([s_ref[0], s_ref[1]], axis=1)
    h = jax.nn.relu(dis * scat + b_ref[...])
    y = dis * jnp.dot(h, w_ref[...],
                      preferred_element_type=jnp.float32,
                      precision=lax.Precision.HIGHEST)
    out_ref[0, :, :] = y[:, :D_H // 2]
    out_ref[1, :, :] = y[:, D_H // 2:]


def _tc2(S1, degT, b1, W2):
    return pl.pallas_call(
        _tc2_body,
        grid=(GRID,),
        in_specs=[
            pl.BlockSpec((2, R, D_H // 2), lambda i: (0, i, 0)),
            _SPEC_DEG,
            pl.BlockSpec((1, D_H), lambda i: (0, 0)),
            pl.BlockSpec((D_H, D_H), lambda i: (0, 0)),
        ],
        out_specs=pl.BlockSpec((2, R, D_H // 2), lambda i: (0, i, 0)),
        out_shape=jax.ShapeDtypeStruct((2, NP, D_H // 2), jnp.float32),
    )(S1, degT, b1, W2)


def _tc3_body(s_ref, deg_ref, b_ref, wmu_ref, wlv_ref, out_ref):
    dis = _dis(deg_ref)
    scat = jnp.concatenate([s_ref[0], s_ref[1]], axis=1)
    h = jax.nn.relu(dis * scat + b_ref[...])
    ymu = dis * jnp.dot(h, wmu_ref[...],
                        preferred_element_type=jnp.float32,
                        precision=lax.Precision.HIGHEST)
    ylv = dis * jnp.dot(h, wlv_ref[...],
                        preferred_element_type=jnp.float32,
                        precision=lax.Precision.HIGHEST)
    out_ref[0, :, :] = jnp.concatenate([ymu, ylv], axis=1)
    out_ref[1, :, :] = jnp.zeros((R, 2 * D_Z), jnp.float32)


def _tc3(S2, degT, b2, Wmu, Wlv):
    return pl.pallas_call(
        _tc3_body,
        grid=(GRID,),
        in_specs=[
            pl.BlockSpec((2, R, D_H // 2), lambda i: (0, i, 0)),
            _SPEC_DEG,
            pl.BlockSpec((1, D_H), lambda i: (0, 0)),
            pl.BlockSpec((D_H, D_Z), lambda i: (0, 0)),
            pl.BlockSpec((D_H, D_Z), lambda i: (0, 0)),
        ],
        out_specs=pl.BlockSpec((2, R, 2 * D_Z), lambda i: (0, i, 0)),
        out_shape=jax.ShapeDtypeStruct((2, NP, 2 * D_Z), jnp.float32),
    )(S2, degT, b2, Wmu, Wlv)


def _tc4_body(s_ref, deg_ref, bmu_ref, blv_ref, mu_ref, lv_ref):
    dis = _dis(deg_ref)
    p = s_ref[0] + s_ref[1]
    mu_ref[...] = dis * p[:, :D_Z] + bmu_ref[...]
    lv_ref[...] = dis * p[:, D_Z:] + blv_ref[...]


def _tc4(S3, degT, bmu, blv):
    return pl.pallas_call(
        _tc4_body,
        grid=(GRID,),
        in_specs=[
            pl.BlockSpec((2, R, 2 * D_Z), lambda i: (0, i, 0)),
            _SPEC_DEG,
            pl.BlockSpec((1, D_Z), lambda i: (0, 0)),
            pl.BlockSpec((1, D_Z), lambda i: (0, 0)),
        ],
        out_specs=[
            pl.BlockSpec((R, D_Z), lambda i: (i, 0)),
            pl.BlockSpec((R, D_Z), lambda i: (i, 0)),
        ],
        out_shape=[
            jax.ShapeDtypeStruct((N, D_Z), jnp.float32),
            jax.ShapeDtypeStruct((N, D_Z), jnp.float32),
        ],
    )(S3, degT, bmu, blv)


def kernel(x, edge_index, W1, b1, W2, b2, Wmu, bmu, Wlv, blv):
    edge_index = edge_index.astype(jnp.int32)
    src = edge_index[0]
    dst = edge_index[1]

    src_off = jnp.stack([src, src + NP]).reshape(NC, NS, NCH, 1, K)
    dst_b = jnp.broadcast_to(dst, (NC, E)).reshape(NC, NS, NCH, 1, K)
    src3 = src.reshape(NC, NS, NCH3, 1, K3)
    dst3 = dst.reshape(NC, NS, NCH3, 1, K3)

    deg = _sc_degree(dst)
    degT = deg.reshape(NC, NP).T

    b1r = b1.reshape(1, D_H)
    b2r = b2.reshape(1, D_H)
    bmur = bmu.reshape(1, D_Z)
    blvr = blv.reshape(1, D_Z)

    T1 = _tc1(x, degT, W1)
    S1 = _sc_pass_cols(T1.reshape(NC * NP, D_H // 2), src_off, dst_b)
    T2 = _tc2(S1, degT, b1r, W2)
    S2 = _sc_pass_cols(T2.reshape(NC * NP, D_H // 2), src_off, dst_b)
    T3 = _tc3(S2, degT, b2r, Wmu, Wlv)
    S3 = _sc_pass_edges(T3.reshape(NC * NP, 2 * D_Z), src3, dst3)
    mu, lv = _tc4(S3, degT, bmur, blvr)
    return (mu, lv)
